# baseline (speedup 1.0000x reference)
import functools

import jax
import jax.numpy as jnp
from jax.experimental import pallas as pl
from jax.experimental.pallas import tpu as pltpu


def _router_body(ui_ref, W1_ref, b1_ref, W2_ref, b2_ref, Wr_ref, br_ref,
                 Wt_ref, bt_ref, Wg1t_ref, bg1_ref, Wo1t_ref, bo1_ref,
                 Wa1t_ref, ba1_ref,
                 w_ref, pg_ref, po_ref, pa_ref):
    h = ui_ref[...]
    W1 = W1_ref[...]
    W2 = W2_ref[...]
    b1 = b1_ref[...]
    b2 = b2_ref[...]
    for _ in range(3):
        m = jnp.tanh(jnp.dot(h, W1, preferred_element_type=jnp.float32) + b1)
        h = h + jnp.tanh(jnp.dot(m, W2, preferred_element_type=jnp.float32) + b2)
    logits = jnp.dot(h, Wr_ref[...], preferred_element_type=jnp.float32) + br_ref[...]
    logits = logits - jnp.max(logits, axis=-1, keepdims=True)
    e = jnp.exp(logits)
    w_ref[...] = e / jnp.sum(e, axis=-1, keepdims=True)
    th = jnp.tanh(jnp.dot(h, Wt_ref[...], preferred_element_type=jnp.float32) + bt_ref[...])
    pg_ref[...] = jnp.dot(th, Wg1t_ref[...], preferred_element_type=jnp.float32) + bg1_ref[...]
    po_ref[...] = jnp.dot(th, Wo1t_ref[...], preferred_element_type=jnp.float32) + bo1_ref[...]
    pa_ref[...] = jnp.dot(th, Wa1t_ref[...], preferred_element_type=jnp.float32) + ba1_ref[...]


def _experts_body(c3_ref, c2_ref, c1_ref,
                  Wg1c_ref, Wo1c_ref, Wa1c_ref,
                  Wg2_ref, bg2_ref, Wo2_ref, bo2_ref, Wa2_ref, ba2_ref,
                  wgsT_ref, bgs_ref, Wo3T_ref, bo3_ref, wa3T_ref, ba3_ref,
                  pg_ref, po_ref, pa_ref, w_ref,
                  sdf_ref, img_ref, aud_ref):
    w = w_ref[0]

    c3 = c3_ref[0]
    h = pg_ref[0]
    h = h + c3[:, 0:1] * Wg1c_ref[0:1, :]
    h = h + c3[:, 1:2] * Wg1c_ref[1:2, :]
    h = h + c3[:, 2:3] * Wg1c_ref[2:3, :]
    h = jnp.maximum(h, 0.0)
    h = jnp.dot(h, Wg2_ref[...], preferred_element_type=jnp.float32) + bg2_ref[...]
    h = jnp.maximum(h, 0.0)
    sdf = jnp.sum(h * wgsT_ref[...], axis=-1, keepdims=True) + bgs_ref[...]
    sdf_ref[0] = sdf * w[0:1, 0:1]

    c2 = c2_ref[0]
    h = po_ref[0]
    h = h + c2[:, 0:1] * Wo1c_ref[0:1, :]
    h = h + c2[:, 1:2] * Wo1c_ref[1:2, :]
    h = jnp.maximum(h, 0.0)
    h = jnp.dot(h, Wo2_ref[...], preferred_element_type=jnp.float32) + bo2_ref[...]
    h = jnp.maximum(h, 0.0)
    img = jnp.concatenate(
        [jnp.sum(h * Wo3T_ref[c:c + 1, :], axis=-1, keepdims=True)
         for c in range(3)], axis=-1) + bo3_ref[...]
    img_ref[0] = jax.nn.sigmoid(img) * w[0:1, 1:2]

    c1 = c1_ref[0]
    h = pa_ref[0]
    h = h + c1[:, 0:1] * Wa1c_ref[0:1, :]
    h = jnp.maximum(h, 0.0)
    h = jnp.dot(h, Wa2_ref[...], preferred_element_type=jnp.float32) + ba2_ref[...]
    h = jnp.maximum(h, 0.0)
    aud = jnp.sum(h * wa3T_ref[...], axis=-1, keepdims=True) + ba3_ref[...]
    aud_ref[0] = jnp.tanh(aud) * w[0:1, 2:3]


@functools.partial(jax.jit, static_argnames=("interpret",))
def kernel(user_intent, coords_3d, coords_2d, coords_1d,
           W1, b1, W2, b2, Wr, br, Wt, bt,
           Wg1, bg1, Wg2, bg2, Wgs, bgs, Wgc, bgc,
           Wo1, bo1, Wo2, bo2, Wo3, bo3,
           Wa1, ba1, Wa2, ba2, Wa3, ba3, interpret=False):
    B, N, _ = coords_3d.shape
    GD = user_intent.shape[1]
    H = Wg2.shape[0]
    TN = 512
    f32 = jnp.float32

    w, pg, po, pa = pl.pallas_call(
        _router_body,
        out_shape=(
            jax.ShapeDtypeStruct((B, 3), f32),
            jax.ShapeDtypeStruct((B, H), f32),
            jax.ShapeDtypeStruct((B, H), f32),
            jax.ShapeDtypeStruct((B, H), f32),
        ),
        interpret=interpret,
    )(user_intent, W1, b1.reshape(1, GD), W2, b2.reshape(1, GD),
      Wr, br.reshape(1, 3), Wt, bt.reshape(1, -1),
      Wg1[3:], bg1.reshape(1, H), Wo1[2:], bo1.reshape(1, H),
      Wa1[1:], ba1.reshape(1, H))

    def const(shape):
        return pl.BlockSpec(shape, lambda b, n: (0, 0))

    row = pl.BlockSpec((1, 1, H), lambda b, n: (b, 0, 0))

    sdf, img, aud = pl.pallas_call(
        _experts_body,
        grid=(B, N // TN),
        in_specs=[
            pl.BlockSpec((1, TN, 3), lambda b, n: (b, n, 0)),
            pl.BlockSpec((1, TN, 2), lambda b, n: (b, n, 0)),
            pl.BlockSpec((1, TN, 1), lambda b, n: (b, n, 0)),
            const((3, H)), const((2, H)), const((1, H)),
            const((H, H)), const((1, H)),
            const((H, H)), const((1, H)),
            const((H, H)), const((1, H)),
            const((1, H)), const((1, 1)),
            const((3, H)), const((1, 3)),
            const((1, H)), const((1, 1)),
            row, row, row,
            pl.BlockSpec((1, 1, 3), lambda b, n: (b, 0, 0)),
        ],
        out_specs=[
            pl.BlockSpec((1, TN, 1), lambda b, n: (b, n, 0)),
            pl.BlockSpec((1, TN, 3), lambda b, n: (b, n, 0)),
            pl.BlockSpec((1, TN, 1), lambda b, n: (b, n, 0)),
        ],
        out_shape=(
            jax.ShapeDtypeStruct((B, N, 1), f32),
            jax.ShapeDtypeStruct((B, N, 3), f32),
            jax.ShapeDtypeStruct((B, N, 1), f32),
        ),
        compiler_params=pltpu.CompilerParams(
            dimension_semantics=("parallel", "parallel")),
        interpret=interpret,
    )(coords_3d, coords_2d, coords_1d,
      Wg1[:3], Wo1[:2], Wa1[:1],
      Wg2, bg2.reshape(1, H), Wo2, bo2.reshape(1, H), Wa2, ba2.reshape(1, H),
      Wgs.T, bgs.reshape(1, 1), Wo3.T, bo3.reshape(1, 3), Wa3.T, ba3.reshape(1, 1),
      pg.reshape(B, 1, H), po.reshape(B, 1, H), pa.reshape(B, 1, H),
      w.reshape(B, 1, 3))

    return (w, sdf, img, aud)


# speedup vs baseline: 1.1947x; 1.1947x over previous
import functools

import jax
import jax.numpy as jnp
from jax.experimental import pallas as pl
from jax.experimental.pallas import tpu as pltpu


def _router_body(ui_ref, W1_ref, b1_ref, W2_ref, b2_ref, Wr_ref, br_ref,
                 Wt_ref, bt_ref, Wg1t_ref, bg1_ref, Wo1t_ref, bo1_ref,
                 Wa1t_ref, ba1_ref,
                 w_ref, pg_ref, po_ref, pa_ref):
    h = ui_ref[...]
    W1 = W1_ref[...]
    W2 = W2_ref[...]
    b1 = b1_ref[...]
    b2 = b2_ref[...]
    for _ in range(3):
        m = jnp.tanh(jnp.dot(h, W1, preferred_element_type=jnp.float32) + b1)
        h = h + jnp.tanh(jnp.dot(m, W2, preferred_element_type=jnp.float32) + b2)
    logits = jnp.dot(h, Wr_ref[...], preferred_element_type=jnp.float32) + br_ref[...]
    logits = logits - jnp.max(logits, axis=-1, keepdims=True)
    e = jnp.exp(logits)
    w_ref[...] = e / jnp.sum(e, axis=-1, keepdims=True)
    th = jnp.tanh(jnp.dot(h, Wt_ref[...], preferred_element_type=jnp.float32) + bt_ref[...])
    pg_ref[...] = jnp.dot(th, Wg1t_ref[...], preferred_element_type=jnp.float32) + bg1_ref[...]
    po_ref[...] = jnp.dot(th, Wo1t_ref[...], preferred_element_type=jnp.float32) + bo1_ref[...]
    pa_ref[...] = jnp.dot(th, Wa1t_ref[...], preferred_element_type=jnp.float32) + ba1_ref[...]


def _experts_body(TN, H, c3_ref, c2_ref, c1_ref, A1_ref,
                  Wg2_ref, bg2_ref, Wo2_ref, bo2_ref, Wa2_ref, ba2_ref,
                  Whead_ref, bhead_ref, w_ref,
                  sdf_ref, img_ref, aud_ref):
    w = w_ref[0]
    ones = jnp.ones((TN, 1), jnp.float32)
    zeros = jnp.zeros((TN, 1), jnp.float32)
    cat = jnp.concatenate(
        [c3_ref[0], c2_ref[0], c1_ref[0], ones, zeros], axis=-1)
    h1 = jnp.dot(cat, A1_ref[0], preferred_element_type=jnp.float32)
    h1 = jnp.maximum(h1, 0.0)

    hg = jnp.dot(h1[:, :H], Wg2_ref[...],
                 preferred_element_type=jnp.float32) + bg2_ref[...]
    ho = jnp.dot(h1[:, H:2 * H], Wo2_ref[...],
                 preferred_element_type=jnp.float32) + bo2_ref[...]
    ha = jnp.dot(h1[:, 2 * H:], Wa2_ref[...],
                 preferred_element_type=jnp.float32) + ba2_ref[...]
    h2 = jnp.maximum(jnp.concatenate([hg, ho, ha], axis=0), 0.0)

    out = jnp.dot(h2, Whead_ref[...],
                  preferred_element_type=jnp.float32) + bhead_ref[...]
    sdf_ref[0] = out[:TN, 0:1] * w[0:1, 0:1]
    img_ref[0] = jax.nn.sigmoid(out[TN:2 * TN, 1:4]) * w[0:1, 1:2]
    aud_ref[0] = jnp.tanh(out[2 * TN:, 4:5]) * w[0:1, 2:3]


@functools.partial(jax.jit, static_argnames=("interpret",))
def kernel(user_intent, coords_3d, coords_2d, coords_1d,
           W1, b1, W2, b2, Wr, br, Wt, bt,
           Wg1, bg1, Wg2, bg2, Wgs, bgs, Wgc, bgc,
           Wo1, bo1, Wo2, bo2, Wo3, bo3,
           Wa1, ba1, Wa2, ba2, Wa3, ba3, interpret=False):
    B, N, _ = coords_3d.shape
    GD = user_intent.shape[1]
    H = Wg2.shape[0]
    TN = 512
    f32 = jnp.float32

    w, pg, po, pa = pl.pallas_call(
        _router_body,
        out_shape=(
            jax.ShapeDtypeStruct((B, 3), f32),
            jax.ShapeDtypeStruct((B, H), f32),
            jax.ShapeDtypeStruct((B, H), f32),
            jax.ShapeDtypeStruct((B, H), f32),
        ),
        interpret=interpret,
    )(user_intent, W1, b1.reshape(1, GD), W2, b2.reshape(1, GD),
      Wr, br.reshape(1, 3), Wt, bt.reshape(1, -1),
      Wg1[3:], bg1.reshape(1, H), Wo1[2:], bo1.reshape(1, H),
      Wa1[1:], ba1.reshape(1, H))

    z = jnp.zeros((B, 1, H), f32)
    zc = jnp.zeros((B, 8, H), f32)
    bcast = lambda W: jnp.broadcast_to(W[None], (B,) + W.shape)
    Ag = jnp.concatenate([bcast(Wg1[:3]), z, z, z, pg[:, None, :], z], axis=1)
    Ao = jnp.concatenate([z, z, z, bcast(Wo1[:2]), z, po[:, None, :], z],
                         axis=1)
    Aa = jnp.concatenate([z, z, z, z, z, bcast(Wa1[:1]), pa[:, None, :], z],
                         axis=1)
    A1 = jnp.concatenate([Ag, Ao, Aa], axis=2)

    Whead = jnp.concatenate([Wgs, Wo3, Wa3, jnp.zeros((H, 3), f32)], axis=1)
    bhead = jnp.concatenate([bgs, bo3, ba3, jnp.zeros((3,), f32)]).reshape(1, 8)

    def const(shape):
        return pl.BlockSpec(shape, lambda b, n: (0, 0))

    sdf, img, aud = pl.pallas_call(
        functools.partial(_experts_body, TN, H),
        grid=(B, N // TN),
        in_specs=[
            pl.BlockSpec((1, TN, 3), lambda b, n: (b, n, 0)),
            pl.BlockSpec((1, TN, 2), lambda b, n: (b, n, 0)),
            pl.BlockSpec((1, TN, 1), lambda b, n: (b, n, 0)),
            pl.BlockSpec((1, 8, 3 * H), lambda b, n: (b, 0, 0)),
            const((H, H)), const((1, H)),
            const((H, H)), const((1, H)),
            const((H, H)), const((1, H)),
            const((H, 8)), const((1, 8)),
            pl.BlockSpec((1, 1, 3), lambda b, n: (b, 0, 0)),
        ],
        out_specs=[
            pl.BlockSpec((1, TN, 1), lambda b, n: (b, n, 0)),
            pl.BlockSpec((1, TN, 3), lambda b, n: (b, n, 0)),
            pl.BlockSpec((1, TN, 1), lambda b, n: (b, n, 0)),
        ],
        out_shape=(
            jax.ShapeDtypeStruct((B, N, 1), f32),
            jax.ShapeDtypeStruct((B, N, 3), f32),
            jax.ShapeDtypeStruct((B, N, 1), f32),
        ),
        compiler_params=pltpu.CompilerParams(
            dimension_semantics=("parallel", "parallel")),
        interpret=interpret,
    )(coords_3d, coords_2d, coords_1d, A1,
      Wg2, bg2.reshape(1, H), Wo2, bo2.reshape(1, H), Wa2, ba2.reshape(1, H),
      Whead, bhead, w.reshape(B, 1, 3))

    return (w, sdf, img, aud)
